# Initial kernel scaffold; baseline (speedup 1.0000x reference)
import jax
import jax.numpy as jnp
from jax.experimental import pallas as pl


def kernel(features, coords):
    raise NotImplementedError("write your pallas kernel here")


# speedup vs baseline: 1.8636x; 1.8636x over previous
import functools

import jax
import jax.numpy as jnp
from jax import lax
from jax.experimental import pallas as pl
from jax.experimental.pallas import tpu as pltpu
from jax.experimental.pallas import tpu_sc as plsc

_R = 32
_B = 8
_C = 64
_N = 65536
_V = _R * _R * _R
_P = 8192


def _coords_body(coords_ref, norm_ref, ids_ref):
    c = coords_ref[...]
    mean = jnp.mean(c, axis=2, keepdims=True)
    n = (c - mean + 1.0) / 2.0
    scaled = jnp.clip(n * float(_R), 0.0, float(_R - 1))
    norm_ref[...] = scaled
    v = jnp.round(scaled).astype(jnp.int32)
    ids_ref[...] = (v[:, 0:1, :] * (_R * _R) + v[:, 1:2, :] * _R
                    + v[:, 2:3, :])


def _coords_call(coords):
    return pl.pallas_call(
        _coords_body,
        grid=(_B,),
        in_specs=[pl.BlockSpec((1, 3, _N), lambda b: (b, 0, 0))],
        out_specs=[
            pl.BlockSpec((1, 3, _N), lambda b: (b, 0, 0)),
            pl.BlockSpec((1, 1, _N), lambda b: (b, 0, 0)),
        ],
        out_shape=[
            jax.ShapeDtypeStruct((_B, 3, _N), jnp.float32),
            jax.ShapeDtypeStruct((_B, 1, _N), jnp.int32),
        ],
    )(coords)


def _sc_body(feat_hbm, ids_hbm, out_hbm, cnt_v, acc_v, idx_v, val_v):
    wid = lax.axis_index("c") * 16 + lax.axis_index("s")
    b = wid // 4
    c0 = (wid % 4) * 16
    zero16 = jnp.zeros((16,), jnp.float32)
    one16 = jnp.full((16,), 1.0, jnp.float32)

    def zero_cnt(j, _):
        cnt_v[pl.ds(j * 16, 16)] = zero16
        return 0
    lax.fori_loop(0, _V // 16, zero_cnt, 0)

    def count_chunk(k, _):
        pltpu.sync_copy(ids_hbm.at[b, pl.ds(k * _P, _P)], idx_v)

        def inner(i, _):
            plsc.addupdate_scatter(cnt_v, [idx_v[pl.ds(i * 16, 16)]], one16)
            return 0
        lax.fori_loop(0, _P // 16, inner, 0)
        return 0
    lax.fori_loop(0, _N // _P, count_chunk, 0)

    def to_inv(j, _):
        s = pl.ds(j * 16, 16)
        cnt_v[s] = 1.0 / jnp.maximum(cnt_v[s], 1.0)
        return 0
    lax.fori_loop(0, _V // 16, to_inv, 0)

    def chan(ci, _):
        c = c0 + ci

        def zacc(j, _):
            acc_v[pl.ds(j * 16, 16)] = zero16
            return 0
        lax.fori_loop(0, _V // 16, zacc, 0)

        def chunk(k, _):
            pltpu.sync_copy(ids_hbm.at[b, pl.ds(k * _P, _P)], idx_v)
            pltpu.sync_copy(feat_hbm.at[b, c, pl.ds(k * _P, _P)], val_v)

            def inner(i, _):
                s = pl.ds(i * 16, 16)
                plsc.addupdate_scatter(acc_v, [idx_v[s]], val_v[s])
                return 0
            lax.fori_loop(0, _P // 16, inner, 0)
            return 0
        lax.fori_loop(0, _N // _P, chunk, 0)

        def scale(j, _):
            s = pl.ds(j * 16, 16)
            acc_v[s] = acc_v[s] * cnt_v[s]
            return 0
        lax.fori_loop(0, _V // 16, scale, 0)
        pltpu.sync_copy(acc_v, out_hbm.at[b, c])
        return 0
    lax.fori_loop(0, 16, chan, 0)


def _sc_call(features, ids):
    mesh = plsc.VectorSubcoreMesh(core_axis_name="c", subcore_axis_name="s")
    f = functools.partial(
        pl.kernel,
        out_type=jax.ShapeDtypeStruct((_B, _C, _V), jnp.float32),
        mesh=mesh,
        compiler_params=pltpu.CompilerParams(needs_layout_passes=False),
        scratch_types=[
            pltpu.VMEM((_V,), jnp.float32),
            pltpu.VMEM((_V,), jnp.float32),
            pltpu.VMEM((_P,), jnp.int32),
            pltpu.VMEM((_P,), jnp.float32),
        ],
    )(_sc_body)
    return f(features, ids)


def kernel(features, coords):
    coords = lax.stop_gradient(coords)
    norm, ids3 = _coords_call(coords)
    ids = ids3.reshape(_B, _N)
    vox = _sc_call(features, ids)
    return vox.reshape(_B, _C, _R, _R, _R), norm


# speedup vs baseline: 2.8232x; 1.5149x over previous
import functools

import jax
import jax.numpy as jnp
from jax import lax
from jax.experimental import pallas as pl
from jax.experimental.pallas import tpu as pltpu
from jax.experimental.pallas import tpu_sc as plsc

_R = 32
_B = 8
_C = 64
_N = 65536
_V = _R * _R * _R
_P = 4096


def _coords_body(coords_ref, norm_ref, ids_ref):
    c = coords_ref[...]
    mean = jnp.mean(c, axis=2, keepdims=True)
    n = (c - mean + 1.0) / 2.0
    scaled = jnp.clip(n * float(_R), 0.0, float(_R - 1))
    norm_ref[...] = scaled
    v = jnp.round(scaled).astype(jnp.int32)
    ids_ref[...] = (v[:, 0:1, :] * (_R * _R) + v[:, 1:2, :] * _R
                    + v[:, 2:3, :])


def _coords_call(coords):
    return pl.pallas_call(
        _coords_body,
        grid=(_B,),
        in_specs=[pl.BlockSpec((1, 3, _N), lambda b: (b, 0, 0))],
        out_specs=[
            pl.BlockSpec((1, 3, _N), lambda b: (b, 0, 0)),
            pl.BlockSpec((1, 1, _N), lambda b: (b, 0, 0)),
        ],
        out_shape=[
            jax.ShapeDtypeStruct((_B, 3, _N), jnp.float32),
            jax.ShapeDtypeStruct((_B, 1, _N), jnp.int32),
        ],
    )(coords)


def _sc_body(feat_hbm, ids_hbm, out_hbm,
             ids_v, acc_v, inv_v, buf0, buf1, sem0, sem1):
    cid = lax.axis_index("c")
    sid = lax.axis_index("s")
    wid = cid * 16 + sid
    b = wid // 4
    c0 = (wid % 4) * 16
    zero16 = jnp.zeros((16,), jnp.float32)
    one16 = jnp.full((16,), 1.0, jnp.float32)

    pltpu.sync_copy(ids_hbm.at[b], ids_v)

    def zero_acc(j, _):
        base = j * 64
        for u in range(4):
            acc_v[pl.ds(base + u * 16, 16)] = zero16
        return 0

    lax.fori_loop(0, _V // 64, zero_acc, 0)

    def count(i, _):
        base = i * 64
        for u in range(4):
            plsc.addupdate_scatter(
                acc_v, [ids_v[pl.ds(base + u * 16, 16)]], one16)
        return 0
    lax.fori_loop(0, _N // 64, count, 0)

    def to_inv(i, _):
        i0 = 1.0 / jnp.maximum(acc_v[pl.ds(i * 32, 16)], 1.0)
        i1 = 1.0 / jnp.maximum(acc_v[pl.ds(i * 32 + 16, 16)], 1.0)
        inv_v[pl.ds(i * 32, 32)] = plsc.pack(
            i0, i1, format=plsc.PackFormat.INTERLEAVED)
        return 0
    lax.fori_loop(0, _V // 32, to_inv, 0)

    def scatter_chunk(buf, base):
        def inner(i, _):
            p = base + i * 64
            q = i * 64
            for u in range(4):
                plsc.addupdate_scatter(
                    acc_v, [ids_v[pl.ds(p + u * 16, 16)]],
                    buf[pl.ds(q + u * 16, 16)])
            return 0
        lax.fori_loop(0, _P // 64, inner, 0)

    def chan(ci, _):
        c = c0 + ci
        lax.fori_loop(0, _V // 64, zero_acc, 0)
        pltpu.async_copy(feat_hbm.at[b, c, pl.ds(0, _P)], buf0, sem0)

        def pair(k2, _):
            base = k2 * (2 * _P)
            pltpu.async_copy(
                feat_hbm.at[b, c, pl.ds(base + _P, _P)], buf1, sem1)
            pltpu.make_async_copy(
                feat_hbm.at[b, c, pl.ds(0, _P)], buf0, sem0).wait()
            scatter_chunk(buf0, base)

            @pl.when(k2 < _N // (2 * _P) - 1)
            def _():
                pltpu.async_copy(
                    feat_hbm.at[b, c, pl.ds(base + 2 * _P, _P)], buf0, sem0)
            pltpu.make_async_copy(
                feat_hbm.at[b, c, pl.ds(0, _P)], buf1, sem1).wait()
            scatter_chunk(buf1, base + _P)
            return 0
        lax.fori_loop(0, _N // (2 * _P), pair, 0)

        def scale(i, _):
            i0, i1 = plsc.unpack(
                inv_v[pl.ds(i * 32, 32)],
                format=plsc.PackFormat.INTERLEAVED)
            s0 = pl.ds(i * 32, 16)
            s1 = pl.ds(i * 32 + 16, 16)
            acc_v[s0] = acc_v[s0] * i0
            acc_v[s1] = acc_v[s1] * i1
            return 0
        lax.fori_loop(0, _V // 32, scale, 0)
        pltpu.sync_copy(acc_v, out_hbm.at[b, c])
        return 0
    lax.fori_loop(0, 16, chan, 0)


def _sc_call(features, ids):
    mesh = plsc.VectorSubcoreMesh(core_axis_name="c", subcore_axis_name="s")
    f = functools.partial(
        pl.kernel,
        out_type=jax.ShapeDtypeStruct((_B, _C, _V), jnp.float32),
        mesh=mesh,
        compiler_params=pltpu.CompilerParams(needs_layout_passes=False),
        scratch_types=[
            pltpu.VMEM((_N,), jnp.int32),
            pltpu.VMEM((_V,), jnp.float32),
            pltpu.VMEM((_V,), jnp.bfloat16),
            pltpu.VMEM((_P,), jnp.float32),
            pltpu.VMEM((_P,), jnp.float32),
            pltpu.SemaphoreType.DMA,
            pltpu.SemaphoreType.DMA,
        ],
    )(_sc_body)
    return f(features, ids)


def kernel(features, coords):
    coords = lax.stop_gradient(coords)
    norm, ids3 = _coords_call(coords)
    ids = ids3.reshape(_B, _N)
    vox = _sc_call(features, ids)
    return vox.reshape(_B, _C, _R, _R, _R), norm
